# baseline (speedup 1.0000x reference)
import jax
import jax.numpy as jnp
from jax import lax
from jax.experimental import pallas as pl
from jax.experimental.pallas import tpu as pltpu
from jax.experimental.pallas import tpu_sc as plsc

VOCAB = 8192
B, T = 16, 512
N_TOK = B * T

_info = plsc.get_sparse_core_info()
NC, NS = _info.num_cores, _info.num_subcores
NW = NC * NS
TOK_PER_W = N_TOK // NW
CH = 8
NH = 2
HD = VOCAB // NH
NCHUNK = TOK_PER_W // CH


def _gather_body(idx_hbm, table_hbm, out_hbm, idx_v, buf0, buf1, g0, g1, w0, w1):
    wid = lax.axis_index("s") * NC + lax.axis_index("c")
    base = wid * TOK_PER_W
    pltpu.sync_copy(idx_hbm.at[pl.ds(base, TOK_PER_W)], idx_v)

    bufs = (buf0, buf1)
    gsems = (g0, g1)
    wsems = (w0, w1)

    def start_gather(g, h, b):
        pltpu.make_async_copy(
            table_hbm.at[idx_v.at[pl.ds(g * CH, CH)], pl.ds(h, 1)], bufs[b], gsems[b]
        ).start()

    def wait_gather(b):
        pltpu.make_async_copy(
            table_hbm.at[idx_v.at[pl.ds(0, CH)], pl.ds(0, 1)], bufs[b], gsems[b]
        ).wait()

    def start_write(g, h, b):
        pltpu.make_async_copy(
            bufs[b], out_hbm.at[pl.ds(base + g * CH, CH), pl.ds(h, 1)], wsems[b]
        ).start()

    def wait_write(b):
        pltpu.make_async_copy(
            bufs[b], out_hbm.at[pl.ds(base, CH), pl.ds(0, 1)], wsems[b]
        ).wait()

    start_gather(0, 0, 0)
    start_gather(0, 1, 1)

    def step(i, carry):
        for b in range(2):
            wait_gather(b)
            start_write(i, b, b)
            wait_write(b)
            start_gather(i + 1, b, b)
        return carry

    lax.fori_loop(0, NCHUNK - 1, step, 0)

    for b in range(2):
        wait_gather(b)
        start_write(NCHUNK - 1, b, b)
        wait_write(b)


@jax.jit
def _gather(idx_flat, table):
    mesh = plsc.VectorSubcoreMesh(core_axis_name="c", subcore_axis_name="s")
    return pl.kernel(
        _gather_body,
        out_type=jax.ShapeDtypeStruct((N_TOK, NH, HD), jnp.float32),
        mesh=mesh,
        scratch_types=[
            pltpu.VMEM((TOK_PER_W,), jnp.int32),
            pltpu.VMEM((CH, 1, HD), jnp.float32),
            pltpu.VMEM((CH, 1, HD), jnp.float32),
            pltpu.SemaphoreType.DMA,
            pltpu.SemaphoreType.DMA,
            pltpu.SemaphoreType.DMA,
            pltpu.SemaphoreType.DMA,
        ],
    )(idx_flat, table.reshape(VOCAB, NH, HD))


def kernel(idx, table):
    idx_flat = idx.reshape(N_TOK).astype(jnp.int32)
    out = _gather(idx_flat, table)
    return out.reshape(B, T, VOCAB)


# speedup vs baseline: 1.1778x; 1.1778x over previous
import jax
import jax.numpy as jnp
from jax import lax
from jax.experimental import pallas as pl
from jax.experimental.pallas import tpu as pltpu
from jax.experimental.pallas import tpu_sc as plsc

VOCAB = 8192
B, T = 16, 512
N_TOK = B * T

_info = plsc.get_sparse_core_info()
NC, NS = _info.num_cores, _info.num_subcores
NW = NC * NS
TOK_PER_W = N_TOK // NW
HD = VOCAB // 2
ROWS_PER_W = 2 * TOK_PER_W
CH = 8
NCHUNK = ROWS_PER_W // CH
L = 16


def _gather_body(idx_hbm, table_hbm, out_hbm, idx_v, idx2_v, buf0, buf1,
                 g0, g1, w0, w1):
    wid = lax.axis_index("s") * NC + lax.axis_index("c")
    base = wid * TOK_PER_W
    pltpu.sync_copy(idx_hbm.at[pl.ds(base, TOK_PER_W)], idx_v)

    lanes = lax.iota(jnp.int32, L)
    for m in range(ROWS_PER_W // L):
        k = lanes + m * L
        src = lax.shift_right_logical(k, 1)
        v = plsc.load_gather(idx_v, [src])
        idx2_v[pl.ds(m * L, L)] = v * 2 + lax.bitwise_and(k, 1)

    bufs = (buf0, buf1)
    gsems = (g0, g1)
    wsems = (w0, w1)
    obase = wid * ROWS_PER_W

    def start_gather(u, b):
        pltpu.make_async_copy(
            table_hbm.at[idx2_v.at[pl.ds(u * CH, CH)]], bufs[b], gsems[b]
        ).start()

    def wait_gather(b):
        pltpu.make_async_copy(
            table_hbm.at[idx2_v.at[pl.ds(0, CH)]], bufs[b], gsems[b]
        ).wait()

    def start_write(u, b):
        pltpu.make_async_copy(
            bufs[b], out_hbm.at[pl.ds(obase + u * CH, CH)], wsems[b]
        ).start()

    def wait_write(b):
        pltpu.make_async_copy(
            bufs[b], out_hbm.at[pl.ds(obase, CH)], wsems[b]
        ).wait()

    start_gather(0, 0)
    start_gather(1, 1)

    def step(i, carry):
        for b in range(2):
            u = 2 * i + b
            wait_gather(b)
            start_write(u, b)
            wait_write(b)
            start_gather(u + 2, b)
        return carry

    lax.fori_loop(0, NCHUNK // 2 - 1, step, 0)

    for b in range(2):
        u = NCHUNK - 2 + b
        wait_gather(b)
        start_write(u, b)
        wait_write(b)


@jax.jit
def _gather(idx_flat, table2):
    mesh = plsc.VectorSubcoreMesh(core_axis_name="c", subcore_axis_name="s")
    return pl.kernel(
        _gather_body,
        out_type=jax.ShapeDtypeStruct((2 * N_TOK, HD), jnp.float32),
        mesh=mesh,
        compiler_params=pltpu.CompilerParams(needs_layout_passes=False),
        scratch_types=[
            pltpu.VMEM((TOK_PER_W,), jnp.int32),
            pltpu.VMEM((ROWS_PER_W,), jnp.int32),
            pltpu.VMEM((CH, HD), jnp.float32),
            pltpu.VMEM((CH, HD), jnp.float32),
            pltpu.SemaphoreType.DMA,
            pltpu.SemaphoreType.DMA,
            pltpu.SemaphoreType.DMA,
            pltpu.SemaphoreType.DMA,
        ],
    )(idx_flat, table2)


def kernel(idx, table):
    idx_flat = idx.reshape(N_TOK).astype(jnp.int32)
    out = _gather(idx_flat, table.reshape(2 * VOCAB, HD))
    return out.reshape(B, T, VOCAB)
